# baseline (speedup 1.0000x reference)
import jax
import jax.numpy as jnp
from jax.experimental import pallas as pl
from jax.experimental.pallas import tpu as pltpu

_ROWS = 128


def _blend_body(w_ref, idx_ref, x_ref, y_ref, o_ref):
    r = idx_ref[:, :]
    sel = r == 0
    w0 = jnp.where(sel, w_ref[0, 0], w_ref[1, 0])
    w1 = jnp.where(sel, w_ref[0, 1], w_ref[1, 1])
    o_ref[:, :] = x_ref[:, :] * w0 + y_ref[:, :] * w1


def kernel(X, Y, reward, W):
    B, S, D = X.shape
    N = B * S
    x2 = X.reshape(N, D)
    y2 = Y.reshape(N, D)
    idx = reward.reshape(N, 1)

    grid = (N // _ROWS,)
    out = pl.pallas_call(
        _blend_body,
        grid=grid,
        in_specs=[
            pl.BlockSpec(memory_space=pltpu.SMEM),
            pl.BlockSpec((_ROWS, 1), lambda i: (i, 0)),
            pl.BlockSpec((_ROWS, D), lambda i: (i, 0)),
            pl.BlockSpec((_ROWS, D), lambda i: (i, 0)),
        ],
        out_specs=pl.BlockSpec((_ROWS, D), lambda i: (i, 0)),
        out_shape=jax.ShapeDtypeStruct((N, D), jnp.float32),
    )(W, idx, x2, y2)
    return out.reshape(B, S, D)


# speedup vs baseline: 1.0322x; 1.0322x over previous
import jax
import jax.numpy as jnp
from jax.experimental import pallas as pl
from jax.experimental.pallas import tpu as pltpu

_ROWS = 256


def _blend_body(w_ref, idx_ref, x_ref, y_ref, o_ref):
    r = idx_ref[:, :]
    sel = r == 0
    w0 = jnp.where(sel, w_ref[0, 0], w_ref[1, 0])
    w1 = jnp.where(sel, w_ref[0, 1], w_ref[1, 1])
    o_ref[:, :] = x_ref[:, :] * w0 + y_ref[:, :] * w1


def kernel(X, Y, reward, W):
    B, S, D = X.shape
    N = B * S
    x2 = X.reshape(N, D)
    y2 = Y.reshape(N, D)
    idx = reward.reshape(N, 1)

    grid = (N // _ROWS,)
    out = pl.pallas_call(
        _blend_body,
        grid=grid,
        in_specs=[
            pl.BlockSpec(memory_space=pltpu.SMEM),
            pl.BlockSpec((_ROWS, 1), lambda i: (i, 0)),
            pl.BlockSpec((_ROWS, D), lambda i: (i, 0)),
            pl.BlockSpec((_ROWS, D), lambda i: (i, 0)),
        ],
        out_specs=pl.BlockSpec((_ROWS, D), lambda i: (i, 0)),
        out_shape=jax.ShapeDtypeStruct((N, D), jnp.float32),
        compiler_params=pltpu.CompilerParams(
            dimension_semantics=("parallel",),
        ),
    )(W, idx, x2, y2)
    return out.reshape(B, S, D)
